# baseline (speedup 1.0000x reference)
import functools

import jax
import jax.numpy as jnp
from jax import lax
from jax.experimental import pallas as pl
from jax.experimental.pallas import tpu as pltpu
from jax.experimental.pallas import tpu_sc as plsc

_NS = 16
_NC = 2
_CH = 128
_G = 8


def _matmul(x, w):
    n = x.shape[0]
    d = w.shape[1]

    def body(x_ref, w_ref, o0_ref, o1_ref):
        r = jnp.dot(x_ref[...], w_ref[...],
                    preferred_element_type=jnp.float32)
        o0_ref[...] = r
        o1_ref[...] = r

    return pl.pallas_call(
        body,
        out_shape=[jax.ShapeDtypeStruct((n, d), jnp.float32)] * 2,
    )(x, w)


def _combine(p, b):
    _, n, d = p.shape
    blk = 2000

    def body(p_ref, b_ref, o_ref):
        o_ref[...] = p_ref[0] + p_ref[1] + b_ref[...]

    return pl.pallas_call(
        body,
        grid=(n // blk,),
        in_specs=[
            pl.BlockSpec((_NC, blk, d), lambda i: (0, i, 0)),
            pl.BlockSpec((1, d), lambda i: (0, 0)),
        ],
        out_specs=pl.BlockSpec((blk, d), lambda i: (i, 0)),
        out_shape=jax.ShapeDtypeStruct((n, d), jnp.float32),
    )(p, b.reshape(1, d))


def _row_chunks(total, step):
    sizes = []
    left = total
    while left > 0:
        sizes.append(min(step, left))
        left -= sizes[-1]
    return sizes


def _spmm_sc(src2, dst2, ew2, sup_a, sup_b):
    n, d = sup_a.shape
    nw = _NC * _NS
    ncw = dst2.shape[0] // nw
    nsc = ncw // _G
    r_tile = (n // _NS) // 8 * 8
    r_last = n - (_NS - 1) * r_tile
    nvec = d // 16

    mesh = plsc.VectorSubcoreMesh(core_axis_name="c", subcore_axis_name="s")

    @functools.partial(
        pl.kernel,
        out_type=jax.ShapeDtypeStruct((_NC, n, d), jnp.float32),
        mesh=mesh,
        scratch_types=[
            pltpu.VMEM((_G, _CH), jnp.int32),
            pltpu.VMEM((_G, _CH), jnp.int32),
            pltpu.VMEM((_G, _CH), jnp.float32),
            pltpu.VMEM((_CH, d), jnp.float32),
            pltpu.VMEM((_CH, d), jnp.float32),
            pltpu.VMEM_SHARED((n, d), jnp.float32),
            pltpu.SemaphoreType.DMA,
            pltpu.SemaphoreType.DMA,
        ],
    )
    def spmm(src_h, dst_h, ew_h, supa_h, supb_h, out_h, sidx, didx, wv,
             rows_a, rows_b, acc, gs_a, gs_b):
        c = lax.axis_index("c")
        s = lax.axis_index("s")
        wid = s * _NC + c
        bufs = (rows_a, rows_b)
        sems = (gs_a, gs_b)

        def scale_rows(g, rows_ref):
            def body(gg, carry):
                w16 = wv[g, pl.ds(gg * 16, 16)]
                for t in range(16):
                    w = w16[t]
                    i = gg * 16 + t
                    for j in range(nvec):
                        sl = (i, pl.ds(16 * j, 16))
                        rows_ref[sl] = rows_ref[sl] * w
                return carry
            lax.fori_loop(0, _CH // 16, body, 0)

        def for_slab(fn):
            @pl.when(s < _NS - 1)
            def _():
                fn(s * r_tile, _row_chunks(r_tile, _CH))

            @pl.when(s == _NS - 1)
            def _():
                fn((_NS - 1) * r_tile, _row_chunks(r_last, _CH))

        zero = jnp.zeros((16,), jnp.float32)

        def zbody(i, carry):
            for j in range(nvec):
                rows_a[i, pl.ds(16 * j, 16)] = zero
            return carry
        lax.fori_loop(0, _CH, zbody, 0)

        def init_fn(r0, sizes):
            off = 0
            for sz in sizes:
                base = pl.multiple_of(r0 + off, 8)
                pltpu.sync_copy(rows_a.at[pl.ds(0, sz)],
                                acc.at[pl.ds(base, sz)])
                off += sz
        for_slab(init_fn)
        plsc.subcore_barrier()

        def edge_loop(sup_h):
            def ebody(kk, carry):
                row0 = pl.multiple_of(wid * ncw + kk * _G, 8)
                pltpu.sync_copy(src_h.at[pl.ds(row0, _G)], sidx)
                pltpu.sync_copy(dst_h.at[pl.ds(row0, _G)], didx)
                pltpu.sync_copy(ew_h.at[pl.ds(row0, _G)], wv)

                descs = [None] * _G
                descs[0] = pltpu.async_copy(sup_h.at[sidx.at[0]], bufs[0],
                                            sems[0])
                for g in range(_G):
                    if g + 1 < _G:
                        descs[g + 1] = pltpu.async_copy(
                            sup_h.at[sidx.at[g + 1]], bufs[(g + 1) % 2],
                            sems[(g + 1) % 2])
                    descs[g].wait()
                    scale_rows(g, bufs[g % 2])
                    pltpu.sync_copy(bufs[g % 2], acc.at[didx.at[g]],
                                    add=True)
                return carry
            lax.fori_loop(0, nsc, ebody, 0)

        @pl.when(c == 0)
        def _():
            edge_loop(supa_h)

        @pl.when(c == 1)
        def _():
            edge_loop(supb_h)
        plsc.subcore_barrier()

        def out_fn(r0, sizes):
            off = 0
            for sz in sizes:
                base = pl.multiple_of(r0 + off, 8)
                pltpu.sync_copy(acc.at[pl.ds(base, sz)],
                                rows_a.at[pl.ds(0, sz)])
                pltpu.sync_copy(rows_a.at[pl.ds(0, sz)],
                                out_h.at[c, pl.ds(base, sz)])
                off += sz
        for_slab(out_fn)

    return spmm(src2, dst2, ew2, sup_a, sup_b)


def kernel(edge_index, edge_weight, input_feature, W, b):
    src = edge_index[0]
    dst = edge_index[1]
    e = src.shape[0]
    nw = _NC * _NS
    ncw = -(-e // (nw * _CH))
    ncw = -(-ncw // _G) * _G
    ep = nw * ncw * _CH
    pad = ep - e
    src2 = jnp.pad(src, (0, pad)).reshape(ep // _CH, _CH)
    dst2 = jnp.pad(dst, (0, pad)).reshape(ep // _CH, _CH)
    ew2 = jnp.pad(edge_weight, (0, pad)).reshape(ep // _CH, _CH)
    sup_a, sup_b = _matmul(input_feature, W)
    partials = _spmm_sc(src2, dst2, ew2, sup_a, sup_b)
    return _combine(partials, b)


# speedup vs baseline: 2.9047x; 2.9047x over previous
import functools

import jax
import jax.numpy as jnp
from jax import lax
from jax.experimental import pallas as pl
from jax.experimental.pallas import tpu as pltpu
from jax.experimental.pallas import tpu_sc as plsc

_NS = 16
_NC = 2
_CH = 128
_G = 8


def _matmul(x, w):
    n = x.shape[0]
    d = w.shape[1]

    def body(x_ref, w_ref, o_ref):
        o_ref[...] = jnp.dot(x_ref[...], w_ref[...],
                             preferred_element_type=jnp.float32)

    return pl.pallas_call(
        body,
        out_shape=jax.ShapeDtypeStruct((n, d), jnp.float32),
    )(x, w)


def _combine(p, b):
    _, n, d = p.shape
    blk = 2000

    def body(p_ref, b_ref, o_ref):
        o_ref[...] = p_ref[0] + p_ref[1] + b_ref[...]

    return pl.pallas_call(
        body,
        grid=(n // blk,),
        in_specs=[
            pl.BlockSpec((_NC, blk, d), lambda i: (0, i, 0)),
            pl.BlockSpec((1, d), lambda i: (0, 0)),
        ],
        out_specs=pl.BlockSpec((blk, d), lambda i: (i, 0)),
        out_shape=jax.ShapeDtypeStruct((n, d), jnp.float32),
    )(p, b.reshape(1, d))


def _row_chunks(total, step):
    sizes = []
    left = total
    while left > 0:
        sizes.append(min(step, left))
        left -= sizes[-1]
    return sizes


def _spmm_sc(src2, dst2, ew2, sup):
    n, d = sup.shape
    nw = _NC * _NS
    ncw = dst2.shape[0] // nw
    nsc = ncw // _G
    r_tile = (n // _NS) // 8 * 8
    r_last = n - (_NS - 1) * r_tile
    nvec = d // 16

    mesh = plsc.VectorSubcoreMesh(core_axis_name="c", subcore_axis_name="s")

    @functools.partial(
        pl.kernel,
        out_type=jax.ShapeDtypeStruct((_NC, n, d), jnp.float32),
        mesh=mesh,
        scratch_types=[
            pltpu.VMEM((_G, _CH), jnp.int32),
            pltpu.VMEM((_G, _CH), jnp.int32),
            pltpu.VMEM((_G, _CH), jnp.float32),
            pltpu.VMEM((_CH, d), jnp.float32),
            pltpu.VMEM((_CH, d), jnp.float32),
            pltpu.VMEM_SHARED((n, d), jnp.float32),
            pltpu.SemaphoreType.DMA,
            pltpu.SemaphoreType.DMA,
        ],
    )
    def spmm(src_h, dst_h, ew_h, sup_h, out_h, sidx, didx, wv,
             rows_a, rows_b, acc, gs_a, gs_b):
        c = lax.axis_index("c")
        s = lax.axis_index("s")
        wid = s * _NC + c
        bufs = (rows_a, rows_b)
        sems = (gs_a, gs_b)

        def scale_rows(g, rows_ref):
            def body(gg, carry):
                w16 = wv[g, pl.ds(gg * 16, 16)]
                for t in range(16):
                    w = w16[t]
                    i = gg * 16 + t
                    for j in range(nvec):
                        sl = (i, pl.ds(16 * j, 16))
                        rows_ref[sl] = rows_ref[sl] * w
                return carry
            lax.fori_loop(0, _CH // 16, body, 0)

        def for_slab(fn):
            @pl.when(s < _NS - 1)
            def _():
                fn(s * r_tile, _row_chunks(r_tile, _CH))

            @pl.when(s == _NS - 1)
            def _():
                fn((_NS - 1) * r_tile, _row_chunks(r_last, _CH))

        zero = jnp.zeros((16,), jnp.float32)

        def zbody(i, carry):
            for j in range(nvec):
                rows_a[i, pl.ds(16 * j, 16)] = zero
            return carry
        lax.fori_loop(0, _CH, zbody, 0)

        def init_fn(r0, sizes):
            off = 0
            for sz in sizes:
                base = pl.multiple_of(r0 + off, 8)
                pltpu.sync_copy(rows_a.at[pl.ds(0, sz)],
                                acc.at[pl.ds(base, sz)])
                off += sz
        for_slab(init_fn)
        plsc.subcore_barrier()

        def ebody(kk, carry):
            row0 = pl.multiple_of(wid * ncw + kk * _G, 8)
            pltpu.sync_copy(src_h.at[pl.ds(row0, _G)], sidx)
            pltpu.sync_copy(dst_h.at[pl.ds(row0, _G)], didx)
            pltpu.sync_copy(ew_h.at[pl.ds(row0, _G)], wv)

            descs = [None] * _G
            descs[0] = pltpu.async_copy(sup_h.at[sidx.at[0]], bufs[0],
                                        sems[0])
            for g in range(_G):
                if g + 1 < _G:
                    descs[g + 1] = pltpu.async_copy(
                        sup_h.at[sidx.at[g + 1]], bufs[(g + 1) % 2],
                        sems[(g + 1) % 2])
                descs[g].wait()
                scale_rows(g, bufs[g % 2])
                pltpu.sync_copy(bufs[g % 2], acc.at[didx.at[g]], add=True)
            return carry
        lax.fori_loop(0, nsc, ebody, 0)
        plsc.subcore_barrier()

        def out_fn(r0, sizes):
            off = 0
            for sz in sizes:
                base = pl.multiple_of(r0 + off, 8)
                pltpu.sync_copy(acc.at[pl.ds(base, sz)],
                                rows_a.at[pl.ds(0, sz)])
                pltpu.sync_copy(rows_a.at[pl.ds(0, sz)],
                                out_h.at[c, pl.ds(base, sz)])
                off += sz
        for_slab(out_fn)

    return spmm(src2, dst2, ew2, sup)


def kernel(edge_index, edge_weight, input_feature, W, b):
    src = edge_index[0]
    dst = edge_index[1]
    e = src.shape[0]
    nw = _NC * _NS
    ncw = -(-e // (nw * _CH))
    ncw = -(-ncw // _G) * _G
    ep = nw * ncw * _CH
    pad = ep - e
    n = input_feature.shape[0]
    pad_idx = jnp.arange(pad, dtype=jnp.int32) % n
    src2 = jnp.concatenate([src, pad_idx]).reshape(ep // _CH, _CH)
    dst2 = jnp.concatenate([dst, pad_idx]).reshape(ep // _CH, _CH)
    ew2 = jnp.pad(edge_weight, (0, pad)).reshape(ep // _CH, _CH)
    sup = _matmul(input_feature, W)
    partials = _spmm_sc(src2, dst2, ew2, sup)
    return _combine(partials, b)


# speedup vs baseline: 2.9113x; 1.0023x over previous
import functools

import jax
import jax.numpy as jnp
from jax import lax
from jax.experimental import pallas as pl
from jax.experimental.pallas import tpu as pltpu
from jax.experimental.pallas import tpu_sc as plsc

_NS = 16
_NC = 2
_CH = 128
_G = 8


def _matmul(x, w):
    n = x.shape[0]
    d = w.shape[1]

    def body(x_ref, w_ref, o_ref):
        o_ref[...] = jnp.dot(x_ref[...], w_ref[...],
                             preferred_element_type=jnp.float32)

    return pl.pallas_call(
        body,
        out_shape=jax.ShapeDtypeStruct((n, d), jnp.float32),
    )(x, w)


def _combine(p, b):
    _, n, d = p.shape
    blk = 2000

    def body(p_ref, b_ref, o_ref):
        o_ref[...] = p_ref[0] + p_ref[1] + b_ref[...]

    return pl.pallas_call(
        body,
        grid=(n // blk,),
        in_specs=[
            pl.BlockSpec((_NC, blk, d), lambda i: (0, i, 0)),
            pl.BlockSpec((1, d), lambda i: (0, 0)),
        ],
        out_specs=pl.BlockSpec((blk, d), lambda i: (i, 0)),
        out_shape=jax.ShapeDtypeStruct((n, d), jnp.float32),
    )(p, b.reshape(1, d))


def _row_chunks(total, step):
    sizes = []
    left = total
    while left > 0:
        sizes.append(min(step, left))
        left -= sizes[-1]
    return sizes


def _spmm_sc(src2, dst2, ew2, sup):
    n, d = sup.shape
    nw = _NC * _NS
    ncw = dst2.shape[0] // nw
    nsc = ncw // _G
    r_tile = (n // _NS) // 8 * 8
    r_last = n - (_NS - 1) * r_tile
    nvec = d // 16

    mesh = plsc.VectorSubcoreMesh(core_axis_name="c", subcore_axis_name="s")

    @functools.partial(
        pl.kernel,
        out_type=jax.ShapeDtypeStruct((_NC, n, d), jnp.float32),
        mesh=mesh,
        scratch_types=[
            pltpu.VMEM((_G, _CH), jnp.int32),
            pltpu.VMEM((_G, _CH), jnp.int32),
            pltpu.VMEM((_G, _CH), jnp.float32),
            pltpu.VMEM((_CH, d), jnp.float32),
            pltpu.VMEM((_CH, d), jnp.float32),
            pltpu.VMEM_SHARED((n, d), jnp.float32),
            pltpu.SemaphoreType.DMA,
            pltpu.SemaphoreType.DMA,
            pltpu.SemaphoreType.DMA,
            pltpu.SemaphoreType.DMA,
        ],
    )
    def spmm(src_h, dst_h, ew_h, sup_h, out_h, sidx, didx, wv,
             rows_a, rows_b, acc, gs_a, gs_b, ss_a, ss_b):
        c = lax.axis_index("c")
        s = lax.axis_index("s")
        wid = s * _NC + c
        bufs = (rows_a, rows_b)
        sems = (gs_a, gs_b)
        ssems = (ss_a, ss_b)

        def scale_rows(g, rows_ref):
            def body(gg, carry):
                w16 = wv[g, pl.ds(gg * 16, 16)]
                for t in range(16):
                    w = w16[t]
                    i = gg * 16 + t
                    for j in range(nvec):
                        sl = (i, pl.ds(16 * j, 16))
                        rows_ref[sl] = rows_ref[sl] * w
                return carry
            lax.fori_loop(0, _CH // 16, body, 0)

        def for_slab(fn):
            @pl.when(s < _NS - 1)
            def _():
                fn(s * r_tile, _row_chunks(r_tile, _CH))

            @pl.when(s == _NS - 1)
            def _():
                fn((_NS - 1) * r_tile, _row_chunks(r_last, _CH))

        zero = jnp.zeros((16,), jnp.float32)

        def zbody(i, carry):
            for j in range(nvec):
                rows_a[i, pl.ds(16 * j, 16)] = zero
            return carry
        lax.fori_loop(0, _CH, zbody, 0)

        def init_fn(r0, sizes):
            off = 0
            for sz in sizes:
                base = pl.multiple_of(r0 + off, 8)
                pltpu.sync_copy(rows_a.at[pl.ds(0, sz)],
                                acc.at[pl.ds(base, sz)])
                off += sz
        for_slab(init_fn)
        plsc.subcore_barrier()

        def ebody(kk, carry):
            row0 = pl.multiple_of(wid * ncw + kk * _G, 8)
            pltpu.sync_copy(src_h.at[pl.ds(row0, _G)], sidx)
            pltpu.sync_copy(dst_h.at[pl.ds(row0, _G)], didx)
            pltpu.sync_copy(ew_h.at[pl.ds(row0, _G)], wv)

            descs = [None] * _G
            sdescs = [None] * _G
            descs[0] = pltpu.async_copy(sup_h.at[sidx.at[0]], bufs[0],
                                        sems[0])
            for g in range(_G):
                if g + 1 < _G:
                    if g >= 1:
                        sdescs[g - 1].wait()
                    descs[g + 1] = pltpu.async_copy(
                        sup_h.at[sidx.at[g + 1]], bufs[(g + 1) % 2],
                        sems[(g + 1) % 2])
                descs[g].wait()
                scale_rows(g, bufs[g % 2])
                sdescs[g] = pltpu.async_copy(
                    bufs[g % 2], acc.at[didx.at[g]], ssems[g % 2],
                    add=True)
            sdescs[_G - 2].wait()
            sdescs[_G - 1].wait()
            return carry
        lax.fori_loop(0, nsc, ebody, 0)
        plsc.subcore_barrier()

        def out_fn(r0, sizes):
            off = 0
            for sz in sizes:
                base = pl.multiple_of(r0 + off, 8)
                pltpu.sync_copy(acc.at[pl.ds(base, sz)],
                                rows_a.at[pl.ds(0, sz)])
                pltpu.sync_copy(rows_a.at[pl.ds(0, sz)],
                                out_h.at[c, pl.ds(base, sz)])
                off += sz
        for_slab(out_fn)

    return spmm(src2, dst2, ew2, sup)


def kernel(edge_index, edge_weight, input_feature, W, b):
    src = edge_index[0]
    dst = edge_index[1]
    e = src.shape[0]
    nw = _NC * _NS
    ncw = -(-e // (nw * _CH))
    ncw = -(-ncw // _G) * _G
    ep = nw * ncw * _CH
    pad = ep - e
    n = input_feature.shape[0]
    pad_idx = jnp.arange(pad, dtype=jnp.int32) % n
    src2 = jnp.concatenate([src, pad_idx]).reshape(ep // _CH, _CH)
    dst2 = jnp.concatenate([dst, pad_idx]).reshape(ep // _CH, _CH)
    ew2 = jnp.pad(edge_weight, (0, pad)).reshape(ep // _CH, _CH)
    sup = _matmul(input_feature, W)
    partials = _spmm_sc(src2, dst2, ew2, sup)
    return _combine(partials, b)


# speedup vs baseline: 3.1326x; 1.0760x over previous
import functools

import jax
import jax.numpy as jnp
from jax import lax
from jax.experimental import pallas as pl
from jax.experimental.pallas import tpu as pltpu
from jax.experimental.pallas import tpu_sc as plsc

_NS = 16
_NC = 2
_CH = 128
_G = 16


def _matmul(x, w):
    n = x.shape[0]
    d = w.shape[1]

    def body(x_ref, w_ref, o_ref):
        o_ref[...] = jnp.dot(x_ref[...], w_ref[...],
                             preferred_element_type=jnp.float32)

    return pl.pallas_call(
        body,
        out_shape=jax.ShapeDtypeStruct((n, d), jnp.float32),
    )(x, w)


def _combine(p, b):
    _, n, d = p.shape
    blk = 2000

    def body(p_ref, b_ref, o_ref):
        o_ref[...] = p_ref[0] + p_ref[1] + b_ref[...]

    return pl.pallas_call(
        body,
        grid=(n // blk,),
        in_specs=[
            pl.BlockSpec((_NC, blk, d), lambda i: (0, i, 0)),
            pl.BlockSpec((1, d), lambda i: (0, 0)),
        ],
        out_specs=pl.BlockSpec((blk, d), lambda i: (i, 0)),
        out_shape=jax.ShapeDtypeStruct((n, d), jnp.float32),
    )(p, b.reshape(1, d))


def _row_chunks(total, step):
    sizes = []
    left = total
    while left > 0:
        sizes.append(min(step, left))
        left -= sizes[-1]
    return sizes


def _spmm_sc(src2, dst2, ew2, sup):
    n, d = sup.shape
    nw = _NC * _NS
    ncw = dst2.shape[0] // nw
    nsc = ncw // _G
    r_tile = (n // _NS) // 8 * 8
    r_last = n - (_NS - 1) * r_tile
    nvec = d // 16

    mesh = plsc.VectorSubcoreMesh(core_axis_name="c", subcore_axis_name="s")

    @functools.partial(
        pl.kernel,
        out_type=jax.ShapeDtypeStruct((_NC, n, d), jnp.float32),
        mesh=mesh,
        scratch_types=[
            pltpu.VMEM((_G, _CH), jnp.int32),
            pltpu.VMEM((_G, _CH), jnp.int32),
            pltpu.VMEM((_G, _CH), jnp.float32),
            pltpu.VMEM((_CH, d), jnp.float32),
            pltpu.VMEM((_CH, d), jnp.float32),
            pltpu.VMEM_SHARED((n, d), jnp.float32),
            pltpu.SemaphoreType.DMA,
            pltpu.SemaphoreType.DMA,
            pltpu.SemaphoreType.DMA,
            pltpu.SemaphoreType.DMA,
        ],
    )
    def spmm(src_h, dst_h, ew_h, sup_h, out_h, sidx, didx, wv,
             rows_a, rows_b, acc, gs_a, gs_b, ss_a, ss_b):
        c = lax.axis_index("c")
        s = lax.axis_index("s")
        wid = s * _NC + c
        bufs = (rows_a, rows_b)
        sems = (gs_a, gs_b)
        ssems = (ss_a, ss_b)

        def scale_rows(g, rows_ref):
            def body(gg, carry):
                w16 = wv[g, pl.ds(gg * 16, 16)]
                for t in range(16):
                    w = w16[t]
                    i = gg * 16 + t
                    for j in range(nvec):
                        sl = (i, pl.ds(16 * j, 16))
                        rows_ref[sl] = rows_ref[sl] * w
                return carry
            lax.fori_loop(0, _CH // 16, body, 0)

        def for_slab(fn):
            @pl.when(s < _NS - 1)
            def _():
                fn(s * r_tile, _row_chunks(r_tile, _CH))

            @pl.when(s == _NS - 1)
            def _():
                fn((_NS - 1) * r_tile, _row_chunks(r_last, _CH))

        zero = jnp.zeros((16,), jnp.float32)

        def zbody(i, carry):
            for j in range(nvec):
                rows_a[i, pl.ds(16 * j, 16)] = zero
            return carry
        lax.fori_loop(0, _CH, zbody, 0)

        def init_fn(r0, sizes):
            off = 0
            for sz in sizes:
                base = pl.multiple_of(r0 + off, 8)
                pltpu.sync_copy(rows_a.at[pl.ds(0, sz)],
                                acc.at[pl.ds(base, sz)])
                off += sz
        for_slab(init_fn)
        plsc.subcore_barrier()

        def ebody(kk, carry):
            row0 = pl.multiple_of(wid * ncw + kk * _G, 8)
            pltpu.sync_copy(src_h.at[pl.ds(row0, _G)], sidx)
            pltpu.sync_copy(dst_h.at[pl.ds(row0, _G)], didx)
            pltpu.sync_copy(ew_h.at[pl.ds(row0, _G)], wv)

            descs = [None] * _G
            sdescs = [None] * _G
            descs[0] = pltpu.async_copy(sup_h.at[sidx.at[0]], bufs[0],
                                        sems[0])
            for g in range(_G):
                if g + 1 < _G:
                    if g >= 1:
                        sdescs[g - 1].wait()
                    descs[g + 1] = pltpu.async_copy(
                        sup_h.at[sidx.at[g + 1]], bufs[(g + 1) % 2],
                        sems[(g + 1) % 2])
                descs[g].wait()
                scale_rows(g, bufs[g % 2])
                sdescs[g] = pltpu.async_copy(
                    bufs[g % 2], acc.at[didx.at[g]], ssems[g % 2],
                    add=True)
            sdescs[_G - 2].wait()
            sdescs[_G - 1].wait()
            return carry
        lax.fori_loop(0, nsc, ebody, 0)
        plsc.subcore_barrier()

        def out_fn(r0, sizes):
            off = 0
            for sz in sizes:
                base = pl.multiple_of(r0 + off, 8)
                pltpu.sync_copy(acc.at[pl.ds(base, sz)],
                                rows_a.at[pl.ds(0, sz)])
                pltpu.sync_copy(rows_a.at[pl.ds(0, sz)],
                                out_h.at[c, pl.ds(base, sz)])
                off += sz
        for_slab(out_fn)

    return spmm(src2, dst2, ew2, sup)


def kernel(edge_index, edge_weight, input_feature, W, b):
    src = edge_index[0]
    dst = edge_index[1]
    e = src.shape[0]
    nw = _NC * _NS
    ncw = -(-e // (nw * _CH))
    ncw = -(-ncw // _G) * _G
    ep = nw * ncw * _CH
    pad = ep - e
    n = input_feature.shape[0]
    pad_idx = jnp.arange(pad, dtype=jnp.int32) % n
    src2 = jnp.concatenate([src, pad_idx]).reshape(ep // _CH, _CH)
    dst2 = jnp.concatenate([dst, pad_idx]).reshape(ep // _CH, _CH)
    ew2 = jnp.pad(edge_weight, (0, pad)).reshape(ep // _CH, _CH)
    sup = _matmul(input_feature, W)
    partials = _spmm_sc(src2, dst2, ew2, sup)
    return _combine(partials, b)
